# baseline (speedup 1.0000x reference)
import jax
import jax.numpy as jnp
from jax import lax
from jax.experimental import pallas as pl
from jax.experimental.pallas import tpu as pltpu
from jax.experimental.pallas import tpu_sc as plsc

N_NODES = 10000
N_EDGES = 320000
D = 128
N_CLASSES = 40

NC = 2
NS = 16
NW = NC * NS
EPT = N_EDGES // NW
K = 80
NCHUNK = EPT // K
N_PAD = 10240
RPT = N_PAD // NS
R = 512
GRID = N_PAD // R


NBUF = 4
NGRP = NCHUNK // NBUF


def _sc_agg(compute_deg):
  mesh = plsc.VectorSubcoreMesh(
      core_axis_name="c", subcore_axis_name="s", num_cores=NC, num_subcores=NS)

  out_type = [jax.ShapeDtypeStruct((NC, N_PAD, D), jnp.float32)]
  scratch = ([pltpu.VMEM((K, D), jnp.float32) for _ in range(NBUF)]
             + [pltpu.VMEM((K,), jnp.int32) for _ in range(2 * NBUF)]
             + [pltpu.SemaphoreType.DMA for _ in range(2 * NBUF)]
             + [pltpu.SemaphoreType.DMA,
                pltpu.VMEM_SHARED((N_PAD, D), jnp.float32)])
  if compute_deg:
    out_type.append(jax.ShapeDtypeStruct((NC, N_PAD), jnp.float32))
    scratch += [pltpu.VMEM((K,), jnp.float32),
                pltpu.VMEM((RPT,), jnp.float32),
                pltpu.VMEM_SHARED((N_PAD,), jnp.float32)]

  def body(table_hbm, src_hbm, dst_hbm, zrows_hbm, ones_hbm, zdeg_hbm, *refs):
    if compute_deg:
      agg_out, deg_out = refs[0], refs[1]
      refs = refs[2:]
      ones_v, zdeg_v, deg_sh = refs[5 * NBUF + 2:]
    else:
      agg_out = refs[0]
      refs = refs[1:]
    rows_v = refs[:NBUF]
    src_v = refs[NBUF:2 * NBUF]
    dst_v = refs[2 * NBUF:3 * NBUF]
    sems = refs[3 * NBUF:4 * NBUF]
    sem_s = refs[4 * NBUF:5 * NBUF]
    sem_idx = refs[5 * NBUF]
    agg_sh = refs[5 * NBUF + 1]

    cid = lax.axis_index("c")
    sid = lax.axis_index("s")
    wid = sid * NC + cid
    row0 = pl.multiple_of(sid * RPT, 8)

    pltpu.sync_copy(zrows_hbm, rows_v[0])
    for r in range(RPT // K):
      pltpu.sync_copy(rows_v[0], agg_sh.at[pl.ds(row0 + r * K, K)])
    if compute_deg:
      pltpu.sync_copy(ones_hbm, ones_v)
      pltpu.sync_copy(zdeg_hbm, zdeg_v)
      pltpu.sync_copy(zdeg_v, deg_sh.at[pl.ds(row0, RPT)])

    plsc.subcore_barrier()

    base = wid * EPT

    toff = base + NGRP * NBUF * K
    pltpu.sync_copy(src_hbm.at[pl.ds(toff, K)], src_v[0])
    pltpu.sync_copy(dst_hbm.at[pl.ds(toff, K)], dst_v[0])
    pltpu.async_copy(table_hbm.at[src_v[0]], rows_v[0], sems[0]).wait()
    pltpu.sync_copy(rows_v[0], agg_sh.at[dst_v[0]], add=True)
    if compute_deg:
      pltpu.sync_copy(ones_v, deg_sh.at[dst_v[0]], add=True)

    for b in range(NBUF):
      pltpu.async_copy(src_hbm.at[pl.ds(base + b * K, K)], src_v[b], sem_idx)

    def group(g, _):
      off = base + g * (NBUF * K)
      noff = off + NBUF * K
      for b in range(NBUF):
        @pl.when(g > 0)
        def _():
          pltpu.make_async_copy(
              rows_v[b], agg_sh.at[dst_v[b]], sem_s[b]).wait()
          if compute_deg:
            pltpu.make_async_copy(
                ones_v, deg_sh.at[dst_v[b]], sem_s[b]).wait()
        pltpu.async_copy(dst_hbm.at[pl.ds(off + b * K, K)], dst_v[b], sem_s[b])

      for b in range(NBUF):
        pltpu.make_async_copy(
            src_hbm.at[pl.ds(off + b * K, K)], src_v[b], sem_idx).wait()
        pltpu.async_copy(table_hbm.at[src_v[b]], rows_v[b], sems[b])

      for b in range(NBUF):
        pltpu.make_async_copy(
            table_hbm.at[src_v[b]], rows_v[b], sems[b]).wait()
        pltpu.make_async_copy(
            dst_hbm.at[pl.ds(off + b * K, K)], dst_v[b], sem_s[b]).wait()
        pltpu.async_copy(rows_v[b], agg_sh.at[dst_v[b]], sem_s[b], add=True)
        if compute_deg:
          pltpu.async_copy(ones_v, deg_sh.at[dst_v[b]], sem_s[b], add=True)

        @pl.when(g < NGRP - 1)
        def _():
          pltpu.async_copy(
              src_hbm.at[pl.ds(noff + b * K, K)], src_v[b], sem_idx)
      return 0
    lax.fori_loop(0, NGRP, group, 0)

    for b in range(NBUF):
      pltpu.make_async_copy(rows_v[b], agg_sh.at[dst_v[b]], sem_s[b]).wait()
      if compute_deg:
        pltpu.make_async_copy(ones_v, deg_sh.at[dst_v[b]], sem_s[b]).wait()

    plsc.subcore_barrier()

    out2d = agg_out.at[cid]
    for r in range(RPT // K):
      off = row0 + r * K
      b = r % NBUF
      pltpu.sync_copy(agg_sh.at[pl.ds(off, K)], rows_v[b])
      pltpu.sync_copy(rows_v[b], out2d.at[pl.ds(off, K)])
    if compute_deg:
      pltpu.sync_copy(deg_sh.at[pl.ds(row0, RPT)], zdeg_v)
      pltpu.sync_copy(zdeg_v, deg_out.at[cid].at[pl.ds(row0, RPT)])

  return pl.kernel(
      body,
      out_type=tuple(out_type) if compute_deg else out_type[0],
      mesh=mesh,
      scratch_types=scratch)


def _tc_mm(x, W, b):
  def body(x_ref, w_ref, b_ref, o_ref):
    o_ref[...] = (jnp.dot(x_ref[...], w_ref[...],
                          preferred_element_type=jnp.float32) + b_ref[...])

  return pl.pallas_call(
      body,
      grid=(GRID,),
      in_specs=[
          pl.BlockSpec((R, D), lambda i: (i, 0)),
          pl.BlockSpec((D, D), lambda i: (0, 0)),
          pl.BlockSpec((1, D), lambda i: (0, 0)),
      ],
      out_specs=pl.BlockSpec((R, D), lambda i: (i, 0)),
      out_shape=jax.ShapeDtypeStruct((N_PAD, D), jnp.float32),
  )(x, W, b)


def _tc_combine(agg_parts, deg_parts, xr, Wl):
  def body(agg_ref, deg_ref, xr_ref, wl_ref, o_ref):
    agg = agg_ref[0] + agg_ref[1]
    deg = deg_ref[0] + deg_ref[1]
    inv = 1.0 / jnp.maximum(deg, 1.0)
    mean = agg * inv
    h = (jnp.dot(mean, wl_ref[...], preferred_element_type=jnp.float32)
         + xr_ref[...])
    o_ref[...] = jnp.maximum(h, 0.0)

  return pl.pallas_call(
      body,
      grid=(GRID,),
      in_specs=[
          pl.BlockSpec((NC, R, D), lambda i: (0, i, 0)),
          pl.BlockSpec((NC, R, 1), lambda i: (0, i, 0)),
          pl.BlockSpec((R, D), lambda i: (i, 0)),
          pl.BlockSpec((D, D), lambda i: (0, 0)),
      ],
      out_specs=pl.BlockSpec((R, D), lambda i: (i, 0)),
      out_shape=jax.ShapeDtypeStruct((N_PAD, D), jnp.float32),
  )(agg_parts, deg_parts, xr, Wl)


def _tc_layer2_head(agg_parts, deg_parts, xr, Wl,
                    Wlin1, blin1, Wlin2, blin2):
  def body(agg_ref, deg_ref, xr_ref, wl_ref,
           wh1_ref, bh1_ref, wh2_ref, bh2_ref, o_ref, acc_ref):
    agg = agg_ref[0] + agg_ref[1]
    deg = deg_ref[0] + deg_ref[1]
    inv = 1.0 / jnp.maximum(deg, 1.0)
    mean = agg * inv
    h = (jnp.dot(mean, wl_ref[...], preferred_element_type=jnp.float32)
         + xr_ref[...])
    h = jnp.maximum(h, 0.0)

    i = pl.program_id(0)
    gi = i * R + lax.broadcasted_iota(jnp.int32, (R, 1), 0)
    h = jnp.where(gi < N_NODES, h, 0.0)

    @pl.when(i == 0)
    def _():
      acc_ref[...] = jnp.zeros_like(acc_ref)

    acc_ref[...] += jnp.sum(h, axis=0, keepdims=True)

    @pl.when(i == GRID - 1)
    def _():
      g = acc_ref[...] / float(N_NODES)
      g = jnp.maximum(
          jnp.dot(g, wh1_ref[...], preferred_element_type=jnp.float32)
          + bh1_ref[...], 0.0)
      logits = (jnp.dot(g, wh2_ref[...], preferred_element_type=jnp.float32)
                + bh2_ref[...])
      m = jnp.max(logits, axis=-1, keepdims=True)
      lse = jnp.log(jnp.sum(jnp.exp(logits - m), axis=-1, keepdims=True)) + m
      o_ref[...] = logits - lse

  return pl.pallas_call(
      body,
      grid=(GRID,),
      in_specs=[
          pl.BlockSpec((NC, R, D), lambda i: (0, i, 0)),
          pl.BlockSpec((NC, R, 1), lambda i: (0, i, 0)),
          pl.BlockSpec((R, D), lambda i: (i, 0)),
          pl.BlockSpec((D, D), lambda i: (0, 0)),
          pl.BlockSpec((D, D), lambda i: (0, 0)),
          pl.BlockSpec((1, D), lambda i: (0, 0)),
          pl.BlockSpec((D, N_CLASSES), lambda i: (0, 0)),
          pl.BlockSpec((1, N_CLASSES), lambda i: (0, 0)),
      ],
      out_specs=pl.BlockSpec((1, N_CLASSES), lambda i: (0, 0)),
      out_shape=jax.ShapeDtypeStruct((1, N_CLASSES), jnp.float32),
      scratch_shapes=[pltpu.VMEM((1, D), jnp.float32)],
  )(agg_parts, deg_parts, xr, Wl, Wlin1, blin1, Wlin2, blin2)


def kernel(x, edge_index, W1l, b1l, W1r, W2l, b2l, W2r,
           Wlin1, blin1, Wlin2, blin2):
  src = edge_index[0].astype(jnp.int32)
  dst = edge_index[1].astype(jnp.int32)
  x_pad = jnp.pad(x, ((0, N_PAD - N_NODES), (0, 0)))
  zrows = jnp.zeros((K, D), jnp.float32)
  ones1 = jnp.ones((K,), jnp.float32)
  zdeg = jnp.zeros((RPT,), jnp.float32)

  sc = _sc_agg(True)
  agg1, deg = sc(x_pad, src, dst, zrows, ones1, zdeg)
  xr1 = _tc_mm(x_pad, W1r, b1l.reshape(1, D))
  deg3 = deg.reshape(NC, N_PAD, 1)
  h1 = _tc_combine(agg1, deg3, xr1, W1l)
  agg2, _ = sc(h1, src, dst, zrows, ones1, zdeg)
  xr2 = _tc_mm(h1, W2r, b2l.reshape(1, D))
  return _tc_layer2_head(agg2, deg3, xr2, W2l,
                         Wlin1, blin1.reshape(1, D),
                         Wlin2, blin2.reshape(1, N_CLASSES))


# speedup vs baseline: 1.0248x; 1.0248x over previous
import jax
import jax.numpy as jnp
from jax import lax
from jax.experimental import pallas as pl
from jax.experimental.pallas import tpu as pltpu
from jax.experimental.pallas import tpu_sc as plsc

N_NODES = 10000
N_EDGES = 320000
D = 128
N_CLASSES = 40

NC = 2
NS = 16
NW = NC * NS
EPT = N_EDGES // NW
K = 80
NCHUNK = EPT // K
N_PAD = 10240
RPT = N_PAD // NS
R = 512
GRID = N_PAD // R


NBUF = 4
NGRP = NCHUNK // NBUF


def _sc_agg(compute_deg):
  mesh = plsc.VectorSubcoreMesh(
      core_axis_name="c", subcore_axis_name="s", num_cores=NC, num_subcores=NS)

  out_type = [jax.ShapeDtypeStruct((NC, N_PAD, D), jnp.float32)]
  scratch = ([pltpu.VMEM((K, D), jnp.float32) for _ in range(NBUF)]
             + [pltpu.VMEM((K,), jnp.int32) for _ in range(2 * NBUF)]
             + [pltpu.SemaphoreType.DMA for _ in range(2 * NBUF)]
             + [pltpu.SemaphoreType.DMA,
                pltpu.VMEM_SHARED((N_PAD, D), jnp.float32)])
  if compute_deg:
    out_type.append(jax.ShapeDtypeStruct((NC, N_PAD), jnp.float32))
    scratch += [pltpu.VMEM((K,), jnp.float32),
                pltpu.VMEM((RPT,), jnp.float32),
                pltpu.VMEM_SHARED((N_PAD,), jnp.float32)]

  def body(table_hbm, src_hbm, dst_hbm, zrows_hbm, ones_hbm, zdeg_hbm, *refs):
    if compute_deg:
      agg_out, deg_out = refs[0], refs[1]
      refs = refs[2:]
      ones_v, zdeg_v, deg_sh = refs[5 * NBUF + 2:]
    else:
      agg_out = refs[0]
      refs = refs[1:]
    rows_v = refs[:NBUF]
    src_v = refs[NBUF:2 * NBUF]
    dst_v = refs[2 * NBUF:3 * NBUF]
    sems = refs[3 * NBUF:4 * NBUF]
    sem_s = refs[4 * NBUF:5 * NBUF]
    sem_idx = refs[5 * NBUF]
    agg_sh = refs[5 * NBUF + 1]

    cid = lax.axis_index("c")
    sid = lax.axis_index("s")
    wid = sid * NC + cid
    row0 = pl.multiple_of(sid * RPT, 8)

    pltpu.sync_copy(zrows_hbm, rows_v[0])
    for r in range(RPT // K):
      pltpu.async_copy(rows_v[0], agg_sh.at[pl.ds(row0 + r * K, K)],
                       sem_idx)
    if compute_deg:
      pltpu.sync_copy(ones_hbm, ones_v)
      pltpu.sync_copy(zdeg_hbm, zdeg_v)
      pltpu.async_copy(zdeg_v, deg_sh.at[pl.ds(row0, RPT)], sem_idx)
    for r in range(RPT // K):
      pltpu.make_async_copy(rows_v[0], agg_sh.at[pl.ds(row0 + r * K, K)],
                            sem_idx).wait()
    if compute_deg:
      pltpu.make_async_copy(zdeg_v, deg_sh.at[pl.ds(row0, RPT)],
                            sem_idx).wait()

    plsc.subcore_barrier()

    base = wid * EPT

    toff = base + NGRP * NBUF * K
    pltpu.sync_copy(src_hbm.at[pl.ds(toff, K)], src_v[0])
    pltpu.sync_copy(dst_hbm.at[pl.ds(toff, K)], dst_v[0])
    pltpu.async_copy(table_hbm.at[src_v[0]], rows_v[0], sems[0]).wait()
    pltpu.sync_copy(rows_v[0], agg_sh.at[dst_v[0]], add=True)
    if compute_deg:
      pltpu.sync_copy(ones_v, deg_sh.at[dst_v[0]], add=True)

    for b in range(NBUF):
      pltpu.async_copy(src_hbm.at[pl.ds(base + b * K, K)], src_v[b], sem_idx)

    def group(g, _):
      off = base + g * (NBUF * K)
      noff = off + NBUF * K
      for b in range(NBUF):
        @pl.when(g > 0)
        def _():
          pltpu.make_async_copy(
              rows_v[b], agg_sh.at[dst_v[b]], sem_s[b]).wait()
          if compute_deg:
            pltpu.make_async_copy(
                ones_v, deg_sh.at[dst_v[b]], sem_s[b]).wait()
        pltpu.async_copy(dst_hbm.at[pl.ds(off + b * K, K)], dst_v[b], sem_s[b])

      for b in range(NBUF):
        pltpu.make_async_copy(
            src_hbm.at[pl.ds(off + b * K, K)], src_v[b], sem_idx).wait()
        pltpu.async_copy(table_hbm.at[src_v[b]], rows_v[b], sems[b])

      for b in range(NBUF):
        pltpu.make_async_copy(
            table_hbm.at[src_v[b]], rows_v[b], sems[b]).wait()
        pltpu.make_async_copy(
            dst_hbm.at[pl.ds(off + b * K, K)], dst_v[b], sem_s[b]).wait()
        pltpu.async_copy(rows_v[b], agg_sh.at[dst_v[b]], sem_s[b], add=True)
        if compute_deg:
          pltpu.async_copy(ones_v, deg_sh.at[dst_v[b]], sem_s[b], add=True)

        @pl.when(g < NGRP - 1)
        def _():
          pltpu.async_copy(
              src_hbm.at[pl.ds(noff + b * K, K)], src_v[b], sem_idx)
      return 0
    lax.fori_loop(0, NGRP, group, 0)

    for b in range(NBUF):
      pltpu.make_async_copy(rows_v[b], agg_sh.at[dst_v[b]], sem_s[b]).wait()
      if compute_deg:
        pltpu.make_async_copy(ones_v, deg_sh.at[dst_v[b]], sem_s[b]).wait()

    plsc.subcore_barrier()

    out2d = agg_out.at[cid]
    nw_ = RPT // K
    offs = [pl.ds(row0 + r * K, K) for r in range(nw_)]
    for r in range(NBUF):
      pltpu.async_copy(agg_sh.at[offs[r]], rows_v[r], sem_s[r])
    if compute_deg:
      pltpu.async_copy(deg_sh.at[pl.ds(row0, RPT)], zdeg_v, sem_idx)
    for r in range(NBUF):
      pltpu.make_async_copy(agg_sh.at[offs[r]], rows_v[r], sem_s[r]).wait()
      pltpu.async_copy(rows_v[r], out2d.at[offs[r]], sems[r])
    if compute_deg:
      pltpu.make_async_copy(deg_sh.at[pl.ds(row0, RPT)], zdeg_v,
                            sem_idx).wait()
      pltpu.async_copy(zdeg_v, deg_out.at[cid].at[pl.ds(row0, RPT)], sem_idx)
    for r in range(NBUF, nw_):
      b = r % NBUF
      pltpu.make_async_copy(rows_v[b], out2d.at[offs[r - NBUF]],
                            sems[b]).wait()
      pltpu.async_copy(agg_sh.at[offs[r]], rows_v[b], sem_s[b])
    for r in range(NBUF, nw_):
      b = r % NBUF
      pltpu.make_async_copy(agg_sh.at[offs[r]], rows_v[b], sem_s[b]).wait()
      pltpu.async_copy(rows_v[b], out2d.at[offs[r]], sems[b])
    for r in range(NBUF, nw_):
      b = r % NBUF
      pltpu.make_async_copy(rows_v[b], out2d.at[offs[r]], sems[b]).wait()
    if compute_deg:
      pltpu.make_async_copy(zdeg_v, deg_out.at[cid].at[pl.ds(row0, RPT)],
                            sem_idx).wait()

  return pl.kernel(
      body,
      out_type=tuple(out_type) if compute_deg else out_type[0],
      mesh=mesh,
      scratch_types=scratch)


def _tc_layer(agg_parts, deg_parts, x, Wl, bl, Wr):
  def body(agg_ref, deg_ref, x_ref, wl_ref, bl_ref, wr_ref, o_ref):
    agg = agg_ref[0] + agg_ref[1]
    deg = deg_ref[0] + deg_ref[1]
    inv = 1.0 / jnp.maximum(deg, 1.0)
    mean = agg * inv
    h = (jnp.dot(mean, wl_ref[...], preferred_element_type=jnp.float32)
         + jnp.dot(x_ref[...], wr_ref[...], preferred_element_type=jnp.float32)
         + bl_ref[...])
    o_ref[...] = jnp.maximum(h, 0.0)

  return pl.pallas_call(
      body,
      grid=(GRID,),
      in_specs=[
          pl.BlockSpec((NC, R, D), lambda i: (0, i, 0)),
          pl.BlockSpec((NC, R, 1), lambda i: (0, i, 0)),
          pl.BlockSpec((R, D), lambda i: (i, 0)),
          pl.BlockSpec((D, D), lambda i: (0, 0)),
          pl.BlockSpec((1, D), lambda i: (0, 0)),
          pl.BlockSpec((D, D), lambda i: (0, 0)),
      ],
      out_specs=pl.BlockSpec((R, D), lambda i: (i, 0)),
      out_shape=jax.ShapeDtypeStruct((N_PAD, D), jnp.float32),
  )(agg_parts, deg_parts, x, Wl, bl, Wr)


def _tc_layer2_head(agg_parts, deg_parts, h1, Wl, bl, Wr,
                    Wlin1, blin1, Wlin2, blin2):
  def body(agg_ref, deg_ref, x_ref, wl_ref, bl_ref, wr_ref,
           wh1_ref, bh1_ref, wh2_ref, bh2_ref, o_ref, acc_ref):
    agg = agg_ref[0] + agg_ref[1]
    deg = deg_ref[0] + deg_ref[1]
    inv = 1.0 / jnp.maximum(deg, 1.0)
    mean = agg * inv
    h = (jnp.dot(mean, wl_ref[...], preferred_element_type=jnp.float32)
         + jnp.dot(x_ref[...], wr_ref[...], preferred_element_type=jnp.float32)
         + bl_ref[...])
    h = jnp.maximum(h, 0.0)

    i = pl.program_id(0)
    gi = i * R + lax.broadcasted_iota(jnp.int32, (R, 1), 0)
    h = jnp.where(gi < N_NODES, h, 0.0)

    @pl.when(i == 0)
    def _():
      acc_ref[...] = jnp.zeros_like(acc_ref)

    acc_ref[...] += jnp.sum(h, axis=0, keepdims=True)

    @pl.when(i == GRID - 1)
    def _():
      g = acc_ref[...] / float(N_NODES)
      g = jnp.maximum(
          jnp.dot(g, wh1_ref[...], preferred_element_type=jnp.float32)
          + bh1_ref[...], 0.0)
      logits = (jnp.dot(g, wh2_ref[...], preferred_element_type=jnp.float32)
                + bh2_ref[...])
      m = jnp.max(logits, axis=-1, keepdims=True)
      lse = jnp.log(jnp.sum(jnp.exp(logits - m), axis=-1, keepdims=True)) + m
      o_ref[...] = logits - lse

  return pl.pallas_call(
      body,
      grid=(GRID,),
      in_specs=[
          pl.BlockSpec((NC, R, D), lambda i: (0, i, 0)),
          pl.BlockSpec((NC, R, 1), lambda i: (0, i, 0)),
          pl.BlockSpec((R, D), lambda i: (i, 0)),
          pl.BlockSpec((D, D), lambda i: (0, 0)),
          pl.BlockSpec((1, D), lambda i: (0, 0)),
          pl.BlockSpec((D, D), lambda i: (0, 0)),
          pl.BlockSpec((D, D), lambda i: (0, 0)),
          pl.BlockSpec((1, D), lambda i: (0, 0)),
          pl.BlockSpec((D, N_CLASSES), lambda i: (0, 0)),
          pl.BlockSpec((1, N_CLASSES), lambda i: (0, 0)),
      ],
      out_specs=pl.BlockSpec((1, N_CLASSES), lambda i: (0, 0)),
      out_shape=jax.ShapeDtypeStruct((1, N_CLASSES), jnp.float32),
      scratch_shapes=[pltpu.VMEM((1, D), jnp.float32)],
  )(agg_parts, deg_parts, h1, Wl, bl, Wr, Wlin1, blin1, Wlin2, blin2)


def kernel(x, edge_index, W1l, b1l, W1r, W2l, b2l, W2r,
           Wlin1, blin1, Wlin2, blin2):
  src = edge_index[0].astype(jnp.int32)
  dst = edge_index[1].astype(jnp.int32)
  x_pad = jnp.pad(x, ((0, N_PAD - N_NODES), (0, 0)))
  zrows = jnp.zeros((K, D), jnp.float32)
  ones1 = jnp.ones((K,), jnp.float32)
  zdeg = jnp.zeros((RPT,), jnp.float32)

  sc = _sc_agg(True)
  agg1, deg = sc(x_pad, src, dst, zrows, ones1, zdeg)
  deg3 = deg.reshape(NC, N_PAD, 1)
  h1 = _tc_layer(agg1, deg3, x_pad, W1l, b1l.reshape(1, D), W1r)
  agg2, _ = sc(h1, src, dst, zrows, ones1, zdeg)
  return _tc_layer2_head(agg2, deg3, h1, W2l, b2l.reshape(1, D), W2r,
                         Wlin1, blin1.reshape(1, D),
                         Wlin2, blin2.reshape(1, N_CLASSES))


# speedup vs baseline: 1.3390x; 1.3065x over previous
import jax
import jax.numpy as jnp
from jax import lax
from jax.experimental import pallas as pl
from jax.experimental.pallas import tpu as pltpu
from jax.experimental.pallas import tpu_sc as plsc

N_NODES = 10000
N_EDGES = 320000
D = 128
N_CLASSES = 40

NC = 2
NS = 16
NW = NC * NS
EPT = N_EDGES // NW
K = 80
NCHUNK = EPT // K
N_PAD = 10240
RPT = N_PAD // NS
R = 512
GRID = N_PAD // R


NBUF = 4
NGRP = NCHUNK // NBUF


def _sc_agg(compute_deg):
  mesh = plsc.VectorSubcoreMesh(
      core_axis_name="c", subcore_axis_name="s", num_cores=NC, num_subcores=NS)

  out_type = [jax.ShapeDtypeStruct((NC, N_PAD, D), jnp.float32)]
  scratch = ([pltpu.VMEM((K, D), jnp.float32) for _ in range(NBUF)]
             + [pltpu.VMEM((K,), jnp.int32) for _ in range(2 * NBUF)]
             + [pltpu.SemaphoreType.DMA for _ in range(2 * NBUF)]
             + [pltpu.SemaphoreType.DMA,
                pltpu.VMEM_SHARED((N_PAD, D), jnp.float32)])
  if compute_deg:
    out_type.append(jax.ShapeDtypeStruct((NC, N_PAD), jnp.float32))
    scratch += [pltpu.VMEM((K,), jnp.float32),
                pltpu.VMEM((RPT,), jnp.float32),
                pltpu.VMEM_SHARED((N_PAD,), jnp.float32)]

  def body(table_hbm, src_hbm, dst_hbm, zrows_hbm, ones_hbm, zdeg_hbm, *refs):
    if compute_deg:
      agg_out, deg_out = refs[0], refs[1]
      refs = refs[2:]
      ones_v, zdeg_v, deg_sh = refs[5 * NBUF + 2:]
    else:
      agg_out = refs[0]
      refs = refs[1:]
    rows_v = refs[:NBUF]
    src_v = refs[NBUF:2 * NBUF]
    dst_v = refs[2 * NBUF:3 * NBUF]
    sems = refs[3 * NBUF:4 * NBUF]
    sem_s = refs[4 * NBUF:5 * NBUF]
    sem_idx = refs[5 * NBUF]
    agg_sh = refs[5 * NBUF + 1]

    cid = lax.axis_index("c")
    sid = lax.axis_index("s")
    wid = sid * NC + cid
    row0 = pl.multiple_of(sid * RPT, 8)

    pltpu.sync_copy(zrows_hbm, rows_v[0])
    for r in range(RPT // K):
      pltpu.async_copy(rows_v[0], agg_sh.at[pl.ds(row0 + r * K, K)],
                       sem_idx)
    if compute_deg:
      pltpu.sync_copy(ones_hbm, ones_v)
      pltpu.sync_copy(zdeg_hbm, zdeg_v)
      pltpu.async_copy(zdeg_v, deg_sh.at[pl.ds(row0, RPT)], sem_idx)
    for r in range(RPT // K):
      pltpu.make_async_copy(rows_v[0], agg_sh.at[pl.ds(row0 + r * K, K)],
                            sem_idx).wait()
    if compute_deg:
      pltpu.make_async_copy(zdeg_v, deg_sh.at[pl.ds(row0, RPT)],
                            sem_idx).wait()

    plsc.subcore_barrier()

    base = wid * EPT

    def wait_scatter(b):
      pltpu.make_async_copy(rows_v[b], agg_sh.at[dst_v[b]], sem_s[b]).wait()
      if compute_deg:
        pltpu.make_async_copy(ones_v, deg_sh.at[dst_v[b]], sem_s[b]).wait()

    def fire_scatter(b):
      pltpu.async_copy(rows_v[b], agg_sh.at[dst_v[b]], sem_s[b], add=True)
      if compute_deg:
        pltpu.async_copy(ones_v, deg_sh.at[dst_v[b]], sem_s[b], add=True)

    def fire_dst(off, b):
      pltpu.async_copy(dst_hbm.at[pl.ds(off, K)], dst_v[b], sem_s[b])

    def wait_dst(off, b):
      pltpu.make_async_copy(dst_hbm.at[pl.ds(off, K)], dst_v[b],
                            sem_s[b]).wait()

    def fire_src(off, b):
      pltpu.async_copy(src_hbm.at[pl.ds(off, K)], src_v[b], sem_idx)

    def wait_src(off, b):
      pltpu.make_async_copy(src_hbm.at[pl.ds(off, K)], src_v[b],
                            sem_idx).wait()

    def fire_gather(b):
      pltpu.async_copy(table_hbm.at[src_v[b]], rows_v[b], sems[b])

    def wait_gather(b):
      pltpu.make_async_copy(table_hbm.at[src_v[b]], rows_v[b],
                            sems[b]).wait()

    pltpu.sync_copy(src_hbm.at[pl.ds(base, K)], src_v[0])
    pltpu.sync_copy(src_hbm.at[pl.ds(base + K, K)], src_v[1])
    fire_dst(base, 0)
    fire_dst(base + K, 1)
    fire_gather(0)
    fire_gather(1)
    fire_src(base + 2 * K, 2)
    fire_src(base + 3 * K, 3)

    LAST = NCHUNK - 1

    def ring(g, _):
      for k in range(4):
        off = base + (4 * g + k) * K
        b, b2 = k, (k + 2) % 4
        off2, off4 = off + 2 * K, off + 4 * K

        if k < 2:
          @pl.when(g > 0)
          def _():
            wait_scatter(b2)
        else:
          wait_scatter(b2)

        if k < 3:
          fire_dst(off2, b2)
          wait_src(off2, b2)
          fire_gather(b2)
        else:
          @pl.when(g < NGRP - 1)
          def _():
            fire_dst(off2, b2)
            wait_src(off2, b2)
            fire_gather(b2)

        wait_gather(b)
        if k == 0:
          fire_src(off4, b)
        else:
          @pl.when(g < NGRP - 1)
          def _():
            fire_src(off4, b)
        wait_dst(off, b)
        fire_scatter(b)
      return 0
    lax.fori_loop(0, NGRP, ring, 0)

    off124 = base + LAST * K
    wait_scatter(2)
    wait_gather(0)
    wait_dst(off124, 0)
    fire_scatter(0)
    wait_scatter(3)
    wait_scatter(0)

    plsc.subcore_barrier()

    out2d = agg_out.at[cid]
    nw_ = RPT // K
    offs = [pl.ds(row0 + r * K, K) for r in range(nw_)]
    for r in range(NBUF):
      pltpu.async_copy(agg_sh.at[offs[r]], rows_v[r], sem_s[r])
    if compute_deg:
      pltpu.async_copy(deg_sh.at[pl.ds(row0, RPT)], zdeg_v, sem_idx)
    for r in range(NBUF):
      pltpu.make_async_copy(agg_sh.at[offs[r]], rows_v[r], sem_s[r]).wait()
      pltpu.async_copy(rows_v[r], out2d.at[offs[r]], sems[r])
    if compute_deg:
      pltpu.make_async_copy(deg_sh.at[pl.ds(row0, RPT)], zdeg_v,
                            sem_idx).wait()
      pltpu.async_copy(zdeg_v, deg_out.at[cid].at[pl.ds(row0, RPT)], sem_idx)
    for r in range(NBUF, nw_):
      b = r % NBUF
      pltpu.make_async_copy(rows_v[b], out2d.at[offs[r - NBUF]],
                            sems[b]).wait()
      pltpu.async_copy(agg_sh.at[offs[r]], rows_v[b], sem_s[b])
    for r in range(NBUF, nw_):
      b = r % NBUF
      pltpu.make_async_copy(agg_sh.at[offs[r]], rows_v[b], sem_s[b]).wait()
      pltpu.async_copy(rows_v[b], out2d.at[offs[r]], sems[b])
    for r in range(NBUF, nw_):
      b = r % NBUF
      pltpu.make_async_copy(rows_v[b], out2d.at[offs[r]], sems[b]).wait()
    if compute_deg:
      pltpu.make_async_copy(zdeg_v, deg_out.at[cid].at[pl.ds(row0, RPT)],
                            sem_idx).wait()

  return pl.kernel(
      body,
      out_type=tuple(out_type) if compute_deg else out_type[0],
      mesh=mesh,
      scratch_types=scratch)


def _tc_layer(agg_parts, deg_parts, x, Wl, bl, Wr):
  def body(agg_ref, deg_ref, x_ref, wl_ref, bl_ref, wr_ref, o_ref):
    agg = agg_ref[0] + agg_ref[1]
    deg = deg_ref[0] + deg_ref[1]
    inv = 1.0 / jnp.maximum(deg, 1.0)
    mean = agg * inv
    h = (jnp.dot(mean, wl_ref[...], preferred_element_type=jnp.float32)
         + jnp.dot(x_ref[...], wr_ref[...], preferred_element_type=jnp.float32)
         + bl_ref[...])
    o_ref[...] = jnp.maximum(h, 0.0)

  return pl.pallas_call(
      body,
      grid=(GRID,),
      in_specs=[
          pl.BlockSpec((NC, R, D), lambda i: (0, i, 0)),
          pl.BlockSpec((NC, R, 1), lambda i: (0, i, 0)),
          pl.BlockSpec((R, D), lambda i: (i, 0)),
          pl.BlockSpec((D, D), lambda i: (0, 0)),
          pl.BlockSpec((1, D), lambda i: (0, 0)),
          pl.BlockSpec((D, D), lambda i: (0, 0)),
      ],
      out_specs=pl.BlockSpec((R, D), lambda i: (i, 0)),
      out_shape=jax.ShapeDtypeStruct((N_PAD, D), jnp.float32),
  )(agg_parts, deg_parts, x, Wl, bl, Wr)


def _tc_layer2_head(agg_parts, deg_parts, h1, Wl, bl, Wr,
                    Wlin1, blin1, Wlin2, blin2):
  def body(agg_ref, deg_ref, x_ref, wl_ref, bl_ref, wr_ref,
           wh1_ref, bh1_ref, wh2_ref, bh2_ref, o_ref, acc_ref):
    agg = agg_ref[0] + agg_ref[1]
    deg = deg_ref[0] + deg_ref[1]
    inv = 1.0 / jnp.maximum(deg, 1.0)
    mean = agg * inv
    h = (jnp.dot(mean, wl_ref[...], preferred_element_type=jnp.float32)
         + jnp.dot(x_ref[...], wr_ref[...], preferred_element_type=jnp.float32)
         + bl_ref[...])
    h = jnp.maximum(h, 0.0)

    i = pl.program_id(0)
    gi = i * R + lax.broadcasted_iota(jnp.int32, (R, 1), 0)
    h = jnp.where(gi < N_NODES, h, 0.0)

    @pl.when(i == 0)
    def _():
      acc_ref[...] = jnp.zeros_like(acc_ref)

    acc_ref[...] += jnp.sum(h, axis=0, keepdims=True)

    @pl.when(i == GRID - 1)
    def _():
      g = acc_ref[...] / float(N_NODES)
      g = jnp.maximum(
          jnp.dot(g, wh1_ref[...], preferred_element_type=jnp.float32)
          + bh1_ref[...], 0.0)
      logits = (jnp.dot(g, wh2_ref[...], preferred_element_type=jnp.float32)
                + bh2_ref[...])
      m = jnp.max(logits, axis=-1, keepdims=True)
      lse = jnp.log(jnp.sum(jnp.exp(logits - m), axis=-1, keepdims=True)) + m
      o_ref[...] = logits - lse

  return pl.pallas_call(
      body,
      grid=(GRID,),
      in_specs=[
          pl.BlockSpec((NC, R, D), lambda i: (0, i, 0)),
          pl.BlockSpec((NC, R, 1), lambda i: (0, i, 0)),
          pl.BlockSpec((R, D), lambda i: (i, 0)),
          pl.BlockSpec((D, D), lambda i: (0, 0)),
          pl.BlockSpec((1, D), lambda i: (0, 0)),
          pl.BlockSpec((D, D), lambda i: (0, 0)),
          pl.BlockSpec((D, D), lambda i: (0, 0)),
          pl.BlockSpec((1, D), lambda i: (0, 0)),
          pl.BlockSpec((D, N_CLASSES), lambda i: (0, 0)),
          pl.BlockSpec((1, N_CLASSES), lambda i: (0, 0)),
      ],
      out_specs=pl.BlockSpec((1, N_CLASSES), lambda i: (0, 0)),
      out_shape=jax.ShapeDtypeStruct((1, N_CLASSES), jnp.float32),
      scratch_shapes=[pltpu.VMEM((1, D), jnp.float32)],
  )(agg_parts, deg_parts, h1, Wl, bl, Wr, Wlin1, blin1, Wlin2, blin2)


def kernel(x, edge_index, W1l, b1l, W1r, W2l, b2l, W2r,
           Wlin1, blin1, Wlin2, blin2):
  src = edge_index[0].astype(jnp.int32)
  dst = edge_index[1].astype(jnp.int32)
  x_pad = jnp.pad(x, ((0, N_PAD - N_NODES), (0, 0)))
  zrows = jnp.zeros((K, D), jnp.float32)
  ones1 = jnp.ones((K,), jnp.float32)
  zdeg = jnp.zeros((RPT,), jnp.float32)

  sc = _sc_agg(True)
  agg1, deg = sc(x_pad, src, dst, zrows, ones1, zdeg)
  deg3 = deg.reshape(NC, N_PAD, 1)
  h1 = _tc_layer(agg1, deg3, x_pad, W1l, b1l.reshape(1, D), W1r)
  agg2, _ = sc(h1, src, dst, zrows, ones1, zdeg)
  return _tc_layer2_head(agg2, deg3, h1, W2l, b2l.reshape(1, D), W2r,
                         Wlin1, blin1.reshape(1, D),
                         Wlin2, blin2.reshape(1, N_CLASSES))


# speedup vs baseline: 1.3899x; 1.0380x over previous
import jax
import jax.numpy as jnp
from jax import lax
from jax.experimental import pallas as pl
from jax.experimental.pallas import tpu as pltpu
from jax.experimental.pallas import tpu_sc as plsc

N_NODES = 10000
N_EDGES = 320000
D = 128
N_CLASSES = 40

NC = 2
NS = 16
NW = NC * NS
EPT = N_EDGES // NW
K = 80
NCHUNK = EPT // K
N_PAD = 10240
RPT = N_PAD // NS
R = 512
GRID = N_PAD // R


NBUF = 4
NGRP = NCHUNK // NBUF


def _sc_agg(compute_deg):
  mesh = plsc.VectorSubcoreMesh(
      core_axis_name="c", subcore_axis_name="s", num_cores=NC, num_subcores=NS)

  out_type = [jax.ShapeDtypeStruct((NC, N_PAD, D), jnp.float32)]
  scratch = ([pltpu.VMEM((K, D), jnp.float32) for _ in range(NBUF)]
             + [pltpu.VMEM((K,), jnp.int32) for _ in range(2 * NBUF)]
             + [pltpu.SemaphoreType.DMA for _ in range(2 * NBUF)]
             + [pltpu.SemaphoreType.DMA,
                pltpu.VMEM_SHARED((N_PAD, D), jnp.float32)])
  if compute_deg:
    out_type.append(jax.ShapeDtypeStruct((NC, N_PAD), jnp.float32))
    scratch += [pltpu.VMEM((K,), jnp.float32),
                pltpu.VMEM((RPT,), jnp.float32),
                pltpu.VMEM_SHARED((N_PAD,), jnp.float32)]

  def body(table_hbm, src_hbm, dst_hbm, zrows_hbm, ones_hbm, zdeg_hbm, *refs):
    if compute_deg:
      agg_out, deg_out = refs[0], refs[1]
      refs = refs[2:]
      ones_v, zdeg_v, deg_sh = refs[5 * NBUF + 2:]
    else:
      agg_out = refs[0]
      refs = refs[1:]
    rows_v = refs[:NBUF]
    src_v = refs[NBUF:2 * NBUF]
    dst_v = refs[2 * NBUF:3 * NBUF]
    sems = refs[3 * NBUF:4 * NBUF]
    sem_s = refs[4 * NBUF:5 * NBUF]
    sem_idx = refs[5 * NBUF]
    agg_sh = refs[5 * NBUF + 1]

    cid = lax.axis_index("c")
    sid = lax.axis_index("s")
    wid = sid * NC + cid
    row0 = pl.multiple_of(sid * RPT, 8)

    pltpu.sync_copy(zrows_hbm, rows_v[0])
    for r in range(RPT // K):
      pltpu.async_copy(rows_v[0], agg_sh.at[pl.ds(row0 + r * K, K)],
                       sem_idx)
    if compute_deg:
      pltpu.sync_copy(ones_hbm, ones_v)
      pltpu.sync_copy(zdeg_hbm, zdeg_v)
      pltpu.async_copy(zdeg_v, deg_sh.at[pl.ds(row0, RPT)], sem_idx)
    for r in range(RPT // K):
      pltpu.make_async_copy(rows_v[0], agg_sh.at[pl.ds(row0 + r * K, K)],
                            sem_idx).wait()
    if compute_deg:
      pltpu.make_async_copy(zdeg_v, deg_sh.at[pl.ds(row0, RPT)],
                            sem_idx).wait()

    plsc.subcore_barrier()

    base = wid * EPT

    def wait_scatter(b):
      pltpu.make_async_copy(rows_v[b], agg_sh.at[dst_v[b]], sem_s[b]).wait()
      if compute_deg:
        pltpu.make_async_copy(ones_v, deg_sh.at[dst_v[b]], sem_s[b]).wait()

    def fire_scatter(b):
      pltpu.async_copy(rows_v[b], agg_sh.at[dst_v[b]], sem_s[b], add=True)
      if compute_deg:
        pltpu.async_copy(ones_v, deg_sh.at[dst_v[b]], sem_s[b], add=True)

    def fire_dst(off, b):
      pltpu.async_copy(dst_hbm.at[pl.ds(off, K)], dst_v[b], sem_s[b])

    def wait_dst(off, b):
      pltpu.make_async_copy(dst_hbm.at[pl.ds(off, K)], dst_v[b],
                            sem_s[b]).wait()

    def fire_src(off, b):
      pltpu.async_copy(src_hbm.at[pl.ds(off, K)], src_v[b], sem_idx)

    def wait_src(off, b):
      pltpu.make_async_copy(src_hbm.at[pl.ds(off, K)], src_v[b],
                            sem_idx).wait()

    def fire_gather(b):
      pltpu.async_copy(table_hbm.at[src_v[b]], rows_v[b], sems[b])

    def wait_gather(b):
      pltpu.make_async_copy(table_hbm.at[src_v[b]], rows_v[b],
                            sems[b]).wait()

    pltpu.sync_copy(src_hbm.at[pl.ds(base, K)], src_v[0])
    pltpu.sync_copy(src_hbm.at[pl.ds(base + K, K)], src_v[1])
    fire_dst(base, 0)
    fire_dst(base + K, 1)
    fire_gather(0)
    fire_gather(1)
    fire_src(base + 2 * K, 2)
    fire_src(base + 3 * K, 3)

    LAST = NCHUNK - 1

    def ring(g, _):
      for k in range(4):
        off = base + (4 * g + k) * K
        b, b2 = k, (k + 2) % 4
        off2, off4 = off + 2 * K, off + 4 * K

        if k < 2:
          @pl.when(g > 0)
          def _():
            wait_scatter(b2)
        else:
          wait_scatter(b2)

        if k < 3:
          fire_dst(off2, b2)
          wait_src(off2, b2)
          fire_gather(b2)
        else:
          @pl.when(g < NGRP - 1)
          def _():
            fire_dst(off2, b2)
            wait_src(off2, b2)
            fire_gather(b2)

        wait_gather(b)
        if k == 0:
          fire_src(off4, b)
        else:
          @pl.when(g < NGRP - 1)
          def _():
            fire_src(off4, b)
        wait_dst(off, b)
        fire_scatter(b)
      return 0
    lax.fori_loop(0, NGRP, ring, 0)

    off124 = base + LAST * K
    wait_scatter(2)
    wait_gather(0)
    wait_dst(off124, 0)
    fire_scatter(0)
    wait_scatter(3)
    wait_scatter(0)

    plsc.subcore_barrier()

    out2d = agg_out.at[cid]
    nw_ = RPT // K
    offs = [pl.ds(row0 + r * K, K) for r in range(nw_)]
    for r in range(NBUF):
      pltpu.async_copy(agg_sh.at[offs[r]], rows_v[r], sem_s[r])
    if compute_deg:
      pltpu.async_copy(deg_sh.at[pl.ds(row0, RPT)], zdeg_v, sem_idx)
    for r in range(NBUF):
      pltpu.make_async_copy(agg_sh.at[offs[r]], rows_v[r], sem_s[r]).wait()
      pltpu.async_copy(rows_v[r], out2d.at[offs[r]], sems[r])
    if compute_deg:
      pltpu.make_async_copy(deg_sh.at[pl.ds(row0, RPT)], zdeg_v,
                            sem_idx).wait()
      pltpu.async_copy(zdeg_v, deg_out.at[cid].at[pl.ds(row0, RPT)], sem_idx)
    for r in range(NBUF, nw_):
      b = r % NBUF
      pltpu.make_async_copy(rows_v[b], out2d.at[offs[r - NBUF]],
                            sems[b]).wait()
      pltpu.async_copy(agg_sh.at[offs[r]], rows_v[b], sem_s[b])
    for r in range(NBUF, nw_):
      b = r % NBUF
      pltpu.make_async_copy(agg_sh.at[offs[r]], rows_v[b], sem_s[b]).wait()
      pltpu.async_copy(rows_v[b], out2d.at[offs[r]], sems[b])
    for r in range(NBUF, nw_):
      b = r % NBUF
      pltpu.make_async_copy(rows_v[b], out2d.at[offs[r]], sems[b]).wait()
    if compute_deg:
      pltpu.make_async_copy(zdeg_v, deg_out.at[cid].at[pl.ds(row0, RPT)],
                            sem_idx).wait()

  return pl.kernel(
      body,
      out_type=tuple(out_type) if compute_deg else out_type[0],
      mesh=mesh,
      scratch_types=scratch)


def _tc_layer(agg_parts, deg_parts, x, Wl, bl, Wr):
  def body(agg_ref, deg_ref, x_ref, wl_ref, bl_ref, wr_ref, o_ref):
    agg = agg_ref[0] + agg_ref[1]
    deg = deg_ref[0] + deg_ref[1]
    inv = (1.0 / jnp.maximum(deg, 1.0)).reshape(R, 1)
    mean = agg * inv
    h = (jnp.dot(mean, wl_ref[...], preferred_element_type=jnp.float32)
         + jnp.dot(x_ref[...], wr_ref[...], preferred_element_type=jnp.float32)
         + bl_ref[...])
    o_ref[...] = jnp.maximum(h, 0.0)

  return pl.pallas_call(
      body,
      grid=(GRID,),
      in_specs=[
          pl.BlockSpec((NC, R, D), lambda i: (0, i, 0)),
          pl.BlockSpec((NC, R), lambda i: (0, i)),
          pl.BlockSpec((R, D), lambda i: (i, 0)),
          pl.BlockSpec((D, D), lambda i: (0, 0)),
          pl.BlockSpec((1, D), lambda i: (0, 0)),
          pl.BlockSpec((D, D), lambda i: (0, 0)),
      ],
      out_specs=pl.BlockSpec((R, D), lambda i: (i, 0)),
      out_shape=jax.ShapeDtypeStruct((N_PAD, D), jnp.float32),
  )(agg_parts, deg_parts, x, Wl, bl, Wr)


def _tc_layer2_head(agg_parts, deg_parts, h1, Wl, bl, Wr,
                    Wlin1, blin1, Wlin2, blin2):
  def body(agg_ref, deg_ref, x_ref, wl_ref, bl_ref, wr_ref,
           wh1_ref, bh1_ref, wh2_ref, bh2_ref, o_ref, acc_ref):
    agg = agg_ref[0] + agg_ref[1]
    deg = deg_ref[0] + deg_ref[1]
    inv = (1.0 / jnp.maximum(deg, 1.0)).reshape(R, 1)
    mean = agg * inv
    h = (jnp.dot(mean, wl_ref[...], preferred_element_type=jnp.float32)
         + jnp.dot(x_ref[...], wr_ref[...], preferred_element_type=jnp.float32)
         + bl_ref[...])
    h = jnp.maximum(h, 0.0)

    i = pl.program_id(0)
    gi = i * R + lax.broadcasted_iota(jnp.int32, (R, 1), 0)
    h = jnp.where(gi < N_NODES, h, 0.0)

    @pl.when(i == 0)
    def _():
      acc_ref[...] = jnp.zeros_like(acc_ref)

    acc_ref[...] += jnp.sum(h, axis=0, keepdims=True)

    @pl.when(i == GRID - 1)
    def _():
      g = acc_ref[...] / float(N_NODES)
      g = jnp.maximum(
          jnp.dot(g, wh1_ref[...], preferred_element_type=jnp.float32)
          + bh1_ref[...], 0.0)
      logits = (jnp.dot(g, wh2_ref[...], preferred_element_type=jnp.float32)
                + bh2_ref[...])
      m = jnp.max(logits, axis=-1, keepdims=True)
      lse = jnp.log(jnp.sum(jnp.exp(logits - m), axis=-1, keepdims=True)) + m
      o_ref[...] = logits - lse

  return pl.pallas_call(
      body,
      grid=(GRID,),
      in_specs=[
          pl.BlockSpec((NC, R, D), lambda i: (0, i, 0)),
          pl.BlockSpec((NC, R), lambda i: (0, i)),
          pl.BlockSpec((R, D), lambda i: (i, 0)),
          pl.BlockSpec((D, D), lambda i: (0, 0)),
          pl.BlockSpec((1, D), lambda i: (0, 0)),
          pl.BlockSpec((D, D), lambda i: (0, 0)),
          pl.BlockSpec((D, D), lambda i: (0, 0)),
          pl.BlockSpec((1, D), lambda i: (0, 0)),
          pl.BlockSpec((D, N_CLASSES), lambda i: (0, 0)),
          pl.BlockSpec((1, N_CLASSES), lambda i: (0, 0)),
      ],
      out_specs=pl.BlockSpec((1, N_CLASSES), lambda i: (0, 0)),
      out_shape=jax.ShapeDtypeStruct((1, N_CLASSES), jnp.float32),
      scratch_shapes=[pltpu.VMEM((1, D), jnp.float32)],
  )(agg_parts, deg_parts, h1, Wl, bl, Wr, Wlin1, blin1, Wlin2, blin2)


def kernel(x, edge_index, W1l, b1l, W1r, W2l, b2l, W2r,
           Wlin1, blin1, Wlin2, blin2):
  src = edge_index[0].astype(jnp.int32)
  dst = edge_index[1].astype(jnp.int32)
  x_pad = jnp.pad(x, ((0, N_PAD - N_NODES), (0, 0)))
  zrows = jnp.zeros((K, D), jnp.float32)
  ones1 = jnp.ones((K,), jnp.float32)
  zdeg = jnp.zeros((RPT,), jnp.float32)

  sc = _sc_agg(True)
  agg1, deg = sc(x_pad, src, dst, zrows, ones1, zdeg)
  h1 = _tc_layer(agg1, deg, x_pad, W1l, b1l.reshape(1, D), W1r)
  agg2, _ = sc(h1, src, dst, zrows, ones1, zdeg)
  return _tc_layer2_head(agg2, deg, h1, W2l, b2l.reshape(1, D), W2r,
                         Wlin1, blin1.reshape(1, D),
                         Wlin2, blin2.reshape(1, N_CLASSES))
